# Initial kernel scaffold; baseline (speedup 1.0000x reference)
import jax
import jax.numpy as jnp
from jax.experimental import pallas as pl


def kernel(x, edge_index, batch, W1, b1, W2, b2, W3, b3):
    raise NotImplementedError("write your pallas kernel here")


# speedup vs baseline: 8.4550x; 8.4550x over previous
import jax
import jax.numpy as jnp
from jax import lax
from jax.experimental import pallas as pl
from jax.experimental.pallas import tpu as pltpu
from jax.experimental.pallas import tpu_sc as plsc

_N = 10000
_E = 320000
_F = 128
_H1 = 256
_C = 16
_G = 64

_NC = 2
_NS = 16
_NW = _NC * _NS
_EPW = _E // _NW
_CH = 80
_NCHUNK = _EPW // _CH
_NPAD = 10240
_RPT = _NPAD // _NS
_HR = _NPAD // _F


def _sc_agg(table, src, dst):
    mesh = plsc.VectorSubcoreMesh(core_axis_name="c", subcore_axis_name="s")
    out_type = jax.ShapeDtypeStruct((_NC, _NPAD, _F), jnp.float32)
    scratch = [
        pltpu.VMEM((_CH,), jnp.int32),
        pltpu.VMEM((_CH,), jnp.int32),
        pltpu.VMEM((_CH, _F), jnp.float32),
        pltpu.VMEM_SHARED((_NPAD, _F), jnp.float32),
        pltpu.SemaphoreType.DMA,
    ]

    def body(table_r, src_r, dst_r, out_r, src_v, dst_v, rows_v, acc, sem):
        c = lax.axis_index("c")
        s = lax.axis_index("s")
        wid = s * _NC + c

        zero16 = jnp.zeros((16,), jnp.float32)

        def zero_rows(i, carry):
            for j in range(_F // 16):
                rows_v[i, pl.ds(j * 16, 16)] = zero16
            return carry
        lax.fori_loop(0, _CH, zero_rows, 0)
        for r in range(_RPT // _CH):
            pltpu.sync_copy(rows_v, acc.at[pl.ds(s * _RPT + r * _CH, _CH)])

        plsc.subcore_barrier()

        def chunk(k, carry):
            off = wid * _EPW + k * _CH
            pltpu.sync_copy(src_r.at[pl.ds(off, _CH)], src_v)
            pltpu.sync_copy(dst_r.at[pl.ds(off, _CH)], dst_v)
            pltpu.async_copy(table_r.at[src_v], rows_v, sem).wait()
            pltpu.sync_copy(rows_v, acc.at[dst_v], add=True)
            return carry
        lax.fori_loop(0, _NCHUNK, chunk, 0)

        plsc.subcore_barrier()

        row0 = s * _RPT
        pltpu.sync_copy(acc.at[pl.ds(row0, _RPT)],
                        out_r.at[c, pl.ds(row0, _RPT)])

    return pl.kernel(body, out_type=out_type, mesh=mesh,
                     scratch_types=tuple(scratch))(table, src, dst)


def _sc_deg(dst):
    mesh = plsc.VectorSubcoreMesh(core_axis_name="c", subcore_axis_name="s")
    out_type = jax.ShapeDtypeStruct((_NC, _NPAD, _F), jnp.float32)
    scratch = [
        pltpu.VMEM((_CH,), jnp.int32),
        pltpu.VMEM((_CH, _F), jnp.float32),
        pltpu.VMEM_SHARED((_NPAD, _F), jnp.float32),
        pltpu.SemaphoreType.DMA,
    ]

    def body(dst_r, out_r, dst_v, ones_v, acc, sem):
        c = lax.axis_index("c")
        s = lax.axis_index("s")
        wid = s * _NC + c

        zero16 = jnp.zeros((16,), jnp.float32)

        def zero_rows(i, carry):
            for j in range(_F // 16):
                ones_v[i, pl.ds(j * 16, 16)] = zero16
            return carry
        lax.fori_loop(0, _CH, zero_rows, 0)
        for r in range(_RPT // _CH):
            pltpu.sync_copy(ones_v, acc.at[pl.ds(s * _RPT + r * _CH, _CH)])

        one16 = zero16 + 1.0

        def fill_ones(i, carry):
            for j in range(_F // 16):
                ones_v[i, pl.ds(j * 16, 16)] = one16
            return carry
        lax.fori_loop(0, _CH, fill_ones, 0)

        plsc.subcore_barrier()

        def chunk(k, carry):
            off = wid * _EPW + k * _CH
            pltpu.sync_copy(dst_r.at[pl.ds(off, _CH)], dst_v)
            pltpu.sync_copy(ones_v, acc.at[dst_v], add=True)
            return carry
        lax.fori_loop(0, _NCHUNK, chunk, 0)

        plsc.subcore_barrier()

        row0 = s * _RPT
        pltpu.sync_copy(acc.at[pl.ds(row0, _RPT)],
                        out_r.at[c, pl.ds(row0, _RPT)])

    return pl.kernel(body, out_type=out_type, mesh=mesh,
                     scratch_types=tuple(scratch))(dst)


_BM = 1000
_NBLK = _N // _BM


def _dinv_from(deg_blk):
    deg = jnp.sum(deg_blk, axis=0)
    return jnp.where(deg > 0.0, lax.rsqrt(jnp.maximum(deg, 1e-12)), 0.0)


def _tc1_body(x_r, a_r, d_r, w1_r, b1_r, w2_r, out_r):
    xa = x_r[...] + a_r[0] + a_r[1]
    h1 = jnp.maximum(
        jnp.dot(xa, w1_r[...], preferred_element_type=jnp.float32) + b1_r[...],
        0.0)
    m2 = jnp.dot(h1, w2_r[...], preferred_element_type=jnp.float32)
    out_r[...] = m2 * _dinv_from(d_r[...])


def _tc1(x, agg1, degp, w1t, b1r, w2t):
    return pl.pallas_call(
        _tc1_body,
        grid=(_NBLK,),
        in_specs=[
            pl.BlockSpec((_BM, _F), lambda i: (i, 0)),
            pl.BlockSpec((_NC, _BM, _F), lambda i: (0, i, 0)),
            pl.BlockSpec((_NC, _BM, 1), lambda i: (0, i, 0)),
            pl.BlockSpec((_F, _H1), lambda i: (0, 0)),
            pl.BlockSpec((1, _H1), lambda i: (0, 0)),
            pl.BlockSpec((_H1, _F), lambda i: (0, 0)),
        ],
        out_specs=pl.BlockSpec((_BM, _F), lambda i: (i, 0)),
        out_shape=jax.ShapeDtypeStruct((_N, _F), jnp.float32),
    )(x, agg1, degp, w1t, b1r, w2t)


_BM2 = 1024


def _tc2_body(a_r, d_r, b2_r, w3_r, out_r):
    h = jnp.maximum((a_r[0] + a_r[1]) * _dinv_from(d_r[...]) + b2_r[...], 0.0)
    out_r[...] = jnp.dot(h, w3_r[...], preferred_element_type=jnp.float32)


def _tc2(agg2, degp, b2r, w3t_pad):
    return pl.pallas_call(
        _tc2_body,
        grid=(_NPAD // _BM2,),
        in_specs=[
            pl.BlockSpec((_NC, _BM2, _F), lambda i: (0, i, 0)),
            pl.BlockSpec((_NC, _BM2, 1), lambda i: (0, i, 0)),
            pl.BlockSpec((1, _F), lambda i: (0, 0)),
            pl.BlockSpec((_F, _F), lambda i: (0, 0)),
        ],
        out_specs=pl.BlockSpec((_BM2, _F), lambda i: (i, 0)),
        out_shape=jax.ShapeDtypeStruct((_NPAD, _F), jnp.float32),
    )(agg2, degp, b2r, w3t_pad)


def _tc3_body(a_r, m3_r, b3_r, bt_r, out_r):
    i = pl.program_id(0)
    h128 = jnp.maximum(a_r[0] + a_r[1] + m3_r[...] + b3_r[...], 0.0)
    h = h128[:, :_C]
    oh = (bt_r[...] == lax.broadcasted_iota(jnp.int32, (_BM, _G), 1)
          ).astype(jnp.float32)
    part = lax.dot_general(oh, h, (((0,), (0,)), ((), ())),
                           preferred_element_type=jnp.float32)

    @pl.when(i == 0)
    def _():
        out_r[...] = part

    @pl.when(i > 0)
    def _():
        out_r[...] += part

    @pl.when(i == _NBLK - 1)
    def _():
        p = out_r[...]
        m = jnp.max(p, axis=1, keepdims=True)
        lse = jnp.log(jnp.sum(jnp.exp(p - m), axis=1, keepdims=True)) + m
        out_r[...] = p - lse


def _tc3(agg3, m3, b3r_pad, batch_c):
    return pl.pallas_call(
        _tc3_body,
        grid=(_NBLK,),
        in_specs=[
            pl.BlockSpec((_NC, _BM, _F), lambda i: (0, i, 0)),
            pl.BlockSpec((_BM, _F), lambda i: (i, 0)),
            pl.BlockSpec((1, _F), lambda i: (0, 0)),
            pl.BlockSpec((_BM, 1), lambda i: (i, 0)),
        ],
        out_specs=pl.BlockSpec((_G, _C), lambda i: (0, 0)),
        out_shape=jax.ShapeDtypeStruct((_G, _C), jnp.float32),
    )(agg3, m3, b3r_pad, batch_c)


def kernel(x, edge_index, batch, W1, b1, W2, b2, W3, b3):
    src = edge_index[0]
    dst = edge_index[1]

    agg1 = _sc_agg(x, src, dst)
    degp = _sc_deg(dst)[:, :, 0:1]
    m2s = _tc1(x, agg1, degp, W1.T, b1.reshape(1, -1), W2.T)
    agg2 = _sc_agg(m2s, src, dst)
    w3t_pad = jnp.zeros((_F, _F), jnp.float32).at[:, :_C].set(W3.T)
    m3 = _tc2(agg2, degp, b2.reshape(1, -1), w3t_pad)
    agg3 = _sc_agg(m3, src, dst)
    b3r_pad = jnp.zeros((1, _F), jnp.float32).at[:, :_C].set(b3.reshape(1, -1))
    return _tc3(agg3, m3, b3r_pad, batch.reshape(-1, 1))


# speedup vs baseline: 15.7851x; 1.8670x over previous
import jax
import jax.numpy as jnp
from jax import lax
from jax.experimental import pallas as pl
from jax.experimental.pallas import tpu as pltpu
from jax.experimental.pallas import tpu_sc as plsc

_N = 10000
_E = 320000
_F = 128
_H1 = 256
_C = 16
_G = 64

_NC = 2
_NS = 16
_NW = _NC * _NS
_EPW = _E // _NW
_CH = 80
_NCHUNK = _EPW // _CH
_NPAD = 10240
_RPT = _NPAD // _NS
_HR = _NPAD // _F


def _sc_agg(table, esd):
    mesh = plsc.VectorSubcoreMesh(core_axis_name="c", subcore_axis_name="s")
    out_type = jax.ShapeDtypeStruct((_NC, _NPAD, _F), jnp.float32)
    scratch = [
        pltpu.VMEM((2, _CH), jnp.int32),
        pltpu.VMEM((2, _CH), jnp.int32),
        pltpu.VMEM((_CH, _F), jnp.float32),
        pltpu.VMEM((_CH, _F), jnp.float32),
        pltpu.VMEM_SHARED((_NPAD, _F), jnp.float32),
        pltpu.SemaphoreType.DMA,
        pltpu.SemaphoreType.DMA,
        pltpu.SemaphoreType.DMA,
        pltpu.SemaphoreType.DMA,
    ]

    def body(table_r, esd_r, out_r, idx_a, idx_b, rows_a, rows_b, acc,
             sem_ia, sem_ib, sem_ga, sem_gb):
        c = lax.axis_index("c")
        s = lax.axis_index("s")
        wid = s * _NC + c

        zero16 = jnp.zeros((16,), jnp.float32)

        def zero_rows(i, carry):
            for j in range(_F // 16):
                rows_a[i, pl.ds(j * 16, 16)] = zero16
            return carry
        lax.fori_loop(0, _CH, zero_rows, 0)
        for r in range(_RPT // _CH):
            pltpu.sync_copy(rows_a, acc.at[pl.ds(s * _RPT + r * _CH, _CH)])

        plsc.subcore_barrier()

        def wait_idx(buf, sem):
            pltpu.make_async_copy(esd_r.at[wid, 0], buf, sem).wait()

        def wait_rows(buf, sem):
            pltpu.make_async_copy(table_r.at[pl.ds(0, _CH)], buf, sem).wait()

        pltpu.sync_copy(esd_r.at[wid, 0], idx_a)
        pltpu.async_copy(table_r.at[idx_a.at[0]], rows_a, sem_ga)
        pltpu.async_copy(esd_r.at[wid, 1], idx_b, sem_ib)

        def chunk2(i, carry):
            k1 = 2 * i + 1
            wait_idx(idx_b, sem_ib)
            wait_rows(rows_a, sem_ga)
            pltpu.async_copy(table_r.at[idx_b.at[0]], rows_b, sem_gb)
            pltpu.sync_copy(rows_a, acc.at[idx_a.at[1]], add=True)
            pltpu.async_copy(esd_r.at[wid, k1 + 1], idx_a, sem_ia)
            wait_rows(rows_b, sem_gb)
            wait_idx(idx_a, sem_ia)
            pltpu.async_copy(table_r.at[idx_a.at[0]], rows_a, sem_ga)
            pltpu.sync_copy(rows_b, acc.at[idx_b.at[1]], add=True)

            @pl.when(k1 + 2 < _NCHUNK)
            def _():
                pltpu.async_copy(esd_r.at[wid, k1 + 2], idx_b, sem_ib)
            return carry
        lax.fori_loop(0, (_NCHUNK - 1) // 2, chunk2, 0)

        wait_rows(rows_a, sem_ga)
        pltpu.sync_copy(rows_a, acc.at[idx_a.at[1]], add=True)

        plsc.subcore_barrier()

        row0 = s * _RPT
        pltpu.sync_copy(acc.at[pl.ds(row0, _RPT)],
                        out_r.at[c, pl.ds(row0, _RPT)])

    return pl.kernel(body, out_type=out_type, mesh=mesh,
                     scratch_types=tuple(scratch))(table, esd)


def _sc_deg(dst3):
    mesh = plsc.VectorSubcoreMesh(core_axis_name="c", subcore_axis_name="s")
    out_type = jax.ShapeDtypeStruct((_NC, _NPAD, _F), jnp.float32)
    scratch = [
        pltpu.VMEM((_NCHUNK, _CH), jnp.int32),
        pltpu.VMEM((_CH, _F), jnp.float32),
        pltpu.VMEM_SHARED((_NPAD, _F), jnp.float32),
        pltpu.SemaphoreType.DMA,
    ]

    def body(dst_r, out_r, dst_v, ones_v, acc, sem):
        c = lax.axis_index("c")
        s = lax.axis_index("s")
        wid = s * _NC + c

        zero16 = jnp.zeros((16,), jnp.float32)

        def zero_rows(i, carry):
            for j in range(_F // 16):
                ones_v[i, pl.ds(j * 16, 16)] = zero16
            return carry
        lax.fori_loop(0, _CH, zero_rows, 0)
        for r in range(_RPT // _CH):
            pltpu.sync_copy(ones_v, acc.at[pl.ds(s * _RPT + r * _CH, _CH)])

        one16 = zero16 + 1.0

        def fill_ones(i, carry):
            for j in range(_F // 16):
                ones_v[i, pl.ds(j * 16, 16)] = one16
            return carry
        lax.fori_loop(0, _CH, fill_ones, 0)

        pltpu.sync_copy(dst_r.at[wid], dst_v)

        plsc.subcore_barrier()

        def chunk(k, carry):
            pltpu.async_copy(ones_v, acc.at[dst_v.at[k]], sem, add=True)
            return carry
        lax.fori_loop(0, _NCHUNK, chunk, 0)

        def drain(k, carry):
            pltpu.make_async_copy(ones_v, acc.at[pl.ds(0, _CH)], sem).wait()
            return carry
        lax.fori_loop(0, _NCHUNK, drain, 0)

        plsc.subcore_barrier()

        row0 = s * _RPT
        pltpu.sync_copy(acc.at[pl.ds(row0, _RPT)],
                        out_r.at[c, pl.ds(row0, _RPT)])

    return pl.kernel(body, out_type=out_type, mesh=mesh,
                     scratch_types=tuple(scratch))(dst3)


_BM = 1000
_NBLK = _N // _BM


def _dinv_from(deg_blk):
    deg = jnp.sum(deg_blk, axis=0)
    return jnp.where(deg > 0.0, lax.rsqrt(jnp.maximum(deg, 1e-12)), 0.0)


def _tc1_body(x_r, a_r, d_r, w1_r, b1_r, w2_r, out_r):
    xa = x_r[...] + a_r[0] + a_r[1]
    h1 = jnp.maximum(
        jnp.dot(xa, w1_r[...], preferred_element_type=jnp.float32) + b1_r[...],
        0.0)
    m2 = jnp.dot(h1, w2_r[...], preferred_element_type=jnp.float32)
    out_r[...] = m2 * _dinv_from(d_r[...])


def _tc1(x, agg1, degp, w1t, b1r, w2t):
    return pl.pallas_call(
        _tc1_body,
        grid=(_NBLK,),
        in_specs=[
            pl.BlockSpec((_BM, _F), lambda i: (i, 0)),
            pl.BlockSpec((_NC, _BM, _F), lambda i: (0, i, 0)),
            pl.BlockSpec((_NC, _BM, 1), lambda i: (0, i, 0)),
            pl.BlockSpec((_F, _H1), lambda i: (0, 0)),
            pl.BlockSpec((1, _H1), lambda i: (0, 0)),
            pl.BlockSpec((_H1, _F), lambda i: (0, 0)),
        ],
        out_specs=pl.BlockSpec((_BM, _F), lambda i: (i, 0)),
        out_shape=jax.ShapeDtypeStruct((_N, _F), jnp.float32),
    )(x, agg1, degp, w1t, b1r, w2t)


_BM2 = 1024


def _tc2_body(a_r, d_r, b2_r, w3_r, out_r):
    h = jnp.maximum((a_r[0] + a_r[1]) * _dinv_from(d_r[...]) + b2_r[...], 0.0)
    out_r[...] = jnp.dot(h, w3_r[...], preferred_element_type=jnp.float32)


def _tc2(agg2, degp, b2r, w3t_pad):
    return pl.pallas_call(
        _tc2_body,
        grid=(_NPAD // _BM2,),
        in_specs=[
            pl.BlockSpec((_NC, _BM2, _F), lambda i: (0, i, 0)),
            pl.BlockSpec((_NC, _BM2, 1), lambda i: (0, i, 0)),
            pl.BlockSpec((1, _F), lambda i: (0, 0)),
            pl.BlockSpec((_F, _F), lambda i: (0, 0)),
        ],
        out_specs=pl.BlockSpec((_BM2, _F), lambda i: (i, 0)),
        out_shape=jax.ShapeDtypeStruct((_NPAD, _F), jnp.float32),
    )(agg2, degp, b2r, w3t_pad)


def _tc3_body(a_r, m3_r, b3_r, bt_r, out_r):
    i = pl.program_id(0)
    h128 = jnp.maximum(a_r[0] + a_r[1] + m3_r[...] + b3_r[...], 0.0)
    h = h128[:, :_C]
    oh = (bt_r[...] == lax.broadcasted_iota(jnp.int32, (_BM, _G), 1)
          ).astype(jnp.float32)
    part = lax.dot_general(oh, h, (((0,), (0,)), ((), ())),
                           preferred_element_type=jnp.float32)

    @pl.when(i == 0)
    def _():
        out_r[...] = part

    @pl.when(i > 0)
    def _():
        out_r[...] += part

    @pl.when(i == _NBLK - 1)
    def _():
        p = out_r[...]
        m = jnp.max(p, axis=1, keepdims=True)
        lse = jnp.log(jnp.sum(jnp.exp(p - m), axis=1, keepdims=True)) + m
        out_r[...] = p - lse


def _tc3(agg3, m3, b3r_pad, batch_c):
    return pl.pallas_call(
        _tc3_body,
        grid=(_NBLK,),
        in_specs=[
            pl.BlockSpec((_NC, _BM, _F), lambda i: (0, i, 0)),
            pl.BlockSpec((_BM, _F), lambda i: (i, 0)),
            pl.BlockSpec((1, _F), lambda i: (0, 0)),
            pl.BlockSpec((_BM, 1), lambda i: (i, 0)),
        ],
        out_specs=pl.BlockSpec((_G, _C), lambda i: (0, 0)),
        out_shape=jax.ShapeDtypeStruct((_G, _C), jnp.float32),
    )(agg3, m3, b3r_pad, batch_c)


def kernel(x, edge_index, batch, W1, b1, W2, b2, W3, b3):
    esd = edge_index.reshape(2, _NW, _NCHUNK, _CH).transpose(1, 2, 0, 3)
    dst3 = edge_index[1].reshape(_NW, _NCHUNK, _CH)

    agg1 = _sc_agg(x, esd)
    degp = _sc_deg(dst3)[:, :, 0:1]
    m2s = _tc1(x, agg1, degp, W1.T, b1.reshape(1, -1), W2.T)
    agg2 = _sc_agg(m2s, esd)
    w3t_pad = jnp.zeros((_F, _F), jnp.float32).at[:, :_C].set(W3.T)
    m3 = _tc2(agg2, degp, b2.reshape(1, -1), w3t_pad)
    agg3 = _sc_agg(m3, esd)
    b3r_pad = jnp.zeros((1, _F), jnp.float32).at[:, :_C].set(b3.reshape(1, -1))
    return _tc3(agg3, m3, b3r_pad, batch.reshape(-1, 1))


# speedup vs baseline: 22.1237x; 1.4016x over previous
import jax
import jax.numpy as jnp
from jax import lax
from jax.experimental import pallas as pl
from jax.experimental.pallas import tpu as pltpu
from jax.experimental.pallas import tpu_sc as plsc

_N = 10000
_E = 320000
_F = 128
_H1 = 256
_C = 16
_G = 64

_NC = 2
_NS = 16
_NW = _NC * _NS
_EPW = _E // _NW
_CH = 80
_NCHUNK = _EPW // _CH
_NPAD = 10240
_RPT = _NPAD // _NS
_HR = _NPAD // _F


def _sc_agg(table, esd):
    mesh = plsc.VectorSubcoreMesh(core_axis_name="c", subcore_axis_name="s")
    out_type = jax.ShapeDtypeStruct((_NC, _NPAD, _F), jnp.float32)
    _DR = 4
    _DI = 5
    _U = 20
    scratch = (
        [pltpu.VMEM((2, _CH), jnp.int32) for _ in range(_DI)]
        + [pltpu.VMEM((_CH, _F), jnp.float32) for _ in range(_DR)]
        + [pltpu.VMEM_SHARED((_NPAD, _F), jnp.float32)]
        + [pltpu.SemaphoreType.DMA] * (_DI + 2 * _DR)
    )

    def body(table_r, esd_r, out_r, *scr):
        idx = scr[0:_DI]
        rows = scr[_DI:_DI + _DR]
        acc = scr[_DI + _DR]
        sem_i = scr[_DI + _DR + 1:2 * _DI + _DR + 1]
        sem_g = scr[2 * _DI + _DR + 1:2 * _DI + 2 * _DR + 1]
        sem_s = scr[2 * _DI + 2 * _DR + 1:2 * _DI + 3 * _DR + 1]
        c = lax.axis_index("c")
        s = lax.axis_index("s")
        wid = s * _NC + c

        zero16 = jnp.zeros((16,), jnp.float32)

        def zero_rows(i, carry):
            for j in range(_F // 16):
                rows[0][i, pl.ds(j * 16, 16)] = zero16
            return carry
        lax.fori_loop(0, _CH, zero_rows, 0)
        for r in range(_RPT // _CH):
            pltpu.sync_copy(rows[0], acc.at[pl.ds(s * _RPT + r * _CH, _CH)])

        plsc.subcore_barrier()

        def wait_i(b):
            pltpu.make_async_copy(esd_r.at[wid, 0], idx[b], sem_i[b]).wait()

        def wait_g(b):
            pltpu.make_async_copy(table_r.at[pl.ds(0, _CH)], rows[b],
                                  sem_g[b]).wait()

        def wait_s(b):
            pltpu.make_async_copy(rows[b], acc.at[pl.ds(0, _CH)],
                                  sem_s[b]).wait()

        def step(k, jr, ji, first):
            wait_g(jr)
            pltpu.async_copy(rows[jr], acc.at[idx[ji].at[1]], sem_s[jr],
                             add=True)
            if not first:
                wait_s((jr + _DR - 1) % _DR)
            return k

        for b in range(_DI - 1):
            pltpu.async_copy(esd_r.at[wid, b], idx[b], sem_i[b])
        for b in range(_DR - 1):
            wait_i(b)
            pltpu.async_copy(table_r.at[idx[b].at[0]], rows[b], sem_g[b])

        def chunk20(i, carry):
            k0 = _U * i
            for j in range(_U):
                k = k0 + j
                jr = j % _DR
                ji = j % _DI
                wait_g(jr)
                pltpu.async_copy(rows[jr], acc.at[idx[ji].at[1]], sem_s[jr],
                                 add=True)
                if j > 0:
                    wait_s((jr + _DR - 1) % _DR)
                else:
                    @pl.when(i > 0)
                    def _():
                        wait_s(_DR - 1)
                pltpu.async_copy(esd_r.at[wid, k + _DR],
                                 idx[(j + _DR) % _DI], sem_i[(j + _DR) % _DI])
                wait_i((j + 3) % _DI)
                pltpu.async_copy(table_r.at[idx[(j + 3) % _DI].at[0]],
                                 rows[(j + 3) % _DR], sem_g[(j + 3) % _DR])
            return carry
        lax.fori_loop(0, _NCHUNK // _U, chunk20, 0)

        base = (_NCHUNK // _U) * _U
        for j in range(_NCHUNK - base):
            k = base + j
            jr = j % _DR
            ji = j % _DI
            wait_g(jr)
            pltpu.async_copy(rows[jr], acc.at[idx[ji].at[1]], sem_s[jr],
                             add=True)
            if j > 0:
                wait_s((jr + _DR - 1) % _DR)
            else:
                wait_s(_DR - 1)
            if k + _DR < _NCHUNK:
                pltpu.async_copy(esd_r.at[wid, k + _DR],
                                 idx[(j + _DR) % _DI], sem_i[(j + _DR) % _DI])
            if k + 3 < _NCHUNK:
                wait_i((j + 3) % _DI)
                pltpu.async_copy(table_r.at[idx[(j + 3) % _DI].at[0]],
                                 rows[(j + 3) % _DR], sem_g[(j + 3) % _DR])
        wait_s((_NCHUNK - 1 - base) % _DR)

        plsc.subcore_barrier()

        row0 = s * _RPT
        pltpu.sync_copy(acc.at[pl.ds(row0, _RPT)],
                        out_r.at[c, pl.ds(row0, _RPT)])

    return pl.kernel(body, out_type=out_type, mesh=mesh,
                     scratch_types=tuple(scratch))(table, esd)


def _sc_deg(dst3):
    mesh = plsc.VectorSubcoreMesh(core_axis_name="c", subcore_axis_name="s")
    out_type = jax.ShapeDtypeStruct((_NC, _NPAD, _F), jnp.float32)
    scratch = [
        pltpu.VMEM((_NCHUNK, _CH), jnp.int32),
        pltpu.VMEM((_CH, _F), jnp.float32),
        pltpu.VMEM_SHARED((_NPAD, _F), jnp.float32),
        pltpu.SemaphoreType.DMA,
    ]

    def body(dst_r, out_r, dst_v, ones_v, acc, sem):
        c = lax.axis_index("c")
        s = lax.axis_index("s")
        wid = s * _NC + c

        zero16 = jnp.zeros((16,), jnp.float32)

        def zero_rows(i, carry):
            for j in range(_F // 16):
                ones_v[i, pl.ds(j * 16, 16)] = zero16
            return carry
        lax.fori_loop(0, _CH, zero_rows, 0)
        for r in range(_RPT // _CH):
            pltpu.sync_copy(ones_v, acc.at[pl.ds(s * _RPT + r * _CH, _CH)])

        one16 = zero16 + 1.0

        def fill_ones(i, carry):
            for j in range(_F // 16):
                ones_v[i, pl.ds(j * 16, 16)] = one16
            return carry
        lax.fori_loop(0, _CH, fill_ones, 0)

        pltpu.sync_copy(dst_r.at[wid], dst_v)

        plsc.subcore_barrier()

        def chunk(k, carry):
            pltpu.async_copy(ones_v, acc.at[dst_v.at[k]], sem, add=True)
            return carry
        lax.fori_loop(0, _NCHUNK, chunk, 0)

        def drain(k, carry):
            pltpu.make_async_copy(ones_v, acc.at[pl.ds(0, _CH)], sem).wait()
            return carry
        lax.fori_loop(0, _NCHUNK, drain, 0)

        plsc.subcore_barrier()

        row0 = s * _RPT
        pltpu.sync_copy(acc.at[pl.ds(row0, _RPT)],
                        out_r.at[c, pl.ds(row0, _RPT)])

    return pl.kernel(body, out_type=out_type, mesh=mesh,
                     scratch_types=tuple(scratch))(dst3)


_BM = 1000
_NBLK = _N // _BM


def _dinv_from(deg_blk):
    deg = (deg_blk[0] + deg_blk[1])[:, 0:1]
    return jnp.where(deg > 0.0, lax.rsqrt(jnp.maximum(deg, 1e-12)), 0.0)


def _tc1a_body(x_r, a_r, w1_r, b1_r, w2_r, out_r):
    xa = x_r[...] + a_r[0] + a_r[1]
    h1 = jnp.maximum(
        jnp.dot(xa, w1_r[...], preferred_element_type=jnp.float32) + b1_r[...],
        0.0)
    out_r[...] = jnp.dot(h1, w2_r[...], preferred_element_type=jnp.float32)


def _tc1a(x, agg1, w1t, b1r, w2t):
    return pl.pallas_call(
        _tc1a_body,
        grid=(_NBLK,),
        in_specs=[
            pl.BlockSpec((_BM, _F), lambda i: (i, 0)),
            pl.BlockSpec((_NC, _BM, _F), lambda i: (0, i, 0)),
            pl.BlockSpec((_F, _H1), lambda i: (0, 0)),
            pl.BlockSpec((1, _H1), lambda i: (0, 0)),
            pl.BlockSpec((_H1, _F), lambda i: (0, 0)),
        ],
        out_specs=pl.BlockSpec((_BM, _F), lambda i: (i, 0)),
        out_shape=jax.ShapeDtypeStruct((_N, _F), jnp.float32),
    )(x, agg1, w1t, b1r, w2t)


def _tc1b_body(m_r, d_r, out_r):
    out_r[...] = m_r[...] * _dinv_from(d_r[...])


def _tc1b(m2, degp):
    return pl.pallas_call(
        _tc1b_body,
        grid=(_NBLK,),
        in_specs=[
            pl.BlockSpec((_BM, _F), lambda i: (i, 0)),
            pl.BlockSpec((_NC, _BM, _F), lambda i: (0, i, 0)),
        ],
        out_specs=pl.BlockSpec((_BM, _F), lambda i: (i, 0)),
        out_shape=jax.ShapeDtypeStruct((_N, _F), jnp.float32),
    )(m2, degp)


_BM2 = 1024


def _tc2_body(a_r, d_r, b2_r, w3_r, out_r):
    h = jnp.maximum((a_r[0] + a_r[1]) * _dinv_from(d_r[...]) + b2_r[...], 0.0)
    out_r[...] = jnp.dot(h, w3_r[...], preferred_element_type=jnp.float32)


def _tc2(agg2, degp, b2r, w3t_pad):
    return pl.pallas_call(
        _tc2_body,
        grid=(_NPAD // _BM2,),
        in_specs=[
            pl.BlockSpec((_NC, _BM2, _F), lambda i: (0, i, 0)),
            pl.BlockSpec((_NC, _BM2, _F), lambda i: (0, i, 0)),
            pl.BlockSpec((1, _F), lambda i: (0, 0)),
            pl.BlockSpec((_F, _F), lambda i: (0, 0)),
        ],
        out_specs=pl.BlockSpec((_BM2, _F), lambda i: (i, 0)),
        out_shape=jax.ShapeDtypeStruct((_NPAD, _F), jnp.float32),
    )(agg2, degp, b2r, w3t_pad)


def _tc3_body(a_r, m3_r, b3_r, bt_r, out_r):
    i = pl.program_id(0)
    h128 = jnp.maximum(a_r[0] + a_r[1] + m3_r[...] + b3_r[...], 0.0)
    h = h128[:, :_C]
    oh = (bt_r[...] == lax.broadcasted_iota(jnp.int32, (_BM, _G), 1)
          ).astype(jnp.float32)
    part = lax.dot_general(oh, h, (((0,), (0,)), ((), ())),
                           preferred_element_type=jnp.float32)

    @pl.when(i == 0)
    def _():
        out_r[...] = part

    @pl.when(i > 0)
    def _():
        out_r[...] += part

    @pl.when(i == _NBLK - 1)
    def _():
        p = out_r[...]
        m = jnp.max(p, axis=1, keepdims=True)
        lse = jnp.log(jnp.sum(jnp.exp(p - m), axis=1, keepdims=True)) + m
        out_r[...] = p - lse


def _tc3(agg3, m3, b3r_pad, batch_c):
    return pl.pallas_call(
        _tc3_body,
        grid=(_NBLK,),
        in_specs=[
            pl.BlockSpec((_NC, _BM, _F), lambda i: (0, i, 0)),
            pl.BlockSpec((_BM, _F), lambda i: (i, 0)),
            pl.BlockSpec((1, _F), lambda i: (0, 0)),
            pl.BlockSpec((_BM, 1), lambda i: (i, 0)),
        ],
        out_specs=pl.BlockSpec((_G, _C), lambda i: (0, 0)),
        out_shape=jax.ShapeDtypeStruct((_G, _C), jnp.float32),
    )(agg3, m3, b3r_pad, batch_c)


def kernel(x, edge_index, batch, W1, b1, W2, b2, W3, b3):
    esd = edge_index.reshape(2, _NW, _NCHUNK, _CH).transpose(1, 2, 0, 3)
    dst3 = edge_index[1].reshape(_NW, _NCHUNK, _CH)

    agg1 = _sc_agg(x, esd)
    degp = _sc_deg(dst3)
    m2 = _tc1a(x, agg1, W1.T, b1.reshape(1, -1), W2.T)
    m2s = _tc1b(m2, degp)
    agg2 = _sc_agg(m2s, esd)
    w3t_pad = jnp.zeros((_F, _F), jnp.float32).at[:, :_C].set(W3.T)
    m3 = _tc2(agg2, degp, b2.reshape(1, -1), w3t_pad)
    agg3 = _sc_agg(m3, esd)
    b3r_pad = jnp.zeros((1, _F), jnp.float32).at[:, :_C].set(b3.reshape(1, -1))
    return _tc3(agg3, m3, b3r_pad, batch.reshape(-1, 1))


# speedup vs baseline: 22.5907x; 1.0211x over previous
import jax
import jax.numpy as jnp
from jax import lax
from jax.experimental import pallas as pl
from jax.experimental.pallas import tpu as pltpu
from jax.experimental.pallas import tpu_sc as plsc

_N = 10000
_E = 320000
_F = 128
_H1 = 256
_C = 16
_G = 64

_NC = 2
_NS = 16
_NW = _NC * _NS
_EPW = _E // _NW
_CH = 80
_NCHUNK = _EPW // _CH
_NPAD = 10240
_RPT = _NPAD // _NS
_HR = _NPAD // _F
_CHD = 128
_EPAD = _NW * 80 * _CHD
_NCHD = 80


def _sc_agg(table, esd):
    mesh = plsc.VectorSubcoreMesh(core_axis_name="c", subcore_axis_name="s")
    out_type = jax.ShapeDtypeStruct((_NC, _NPAD, _F), jnp.float32)
    _D = 4
    scratch = (
        [pltpu.VMEM((2, _CH), jnp.int32) for _ in range(_D)]
        + [pltpu.VMEM((_CH, _F), jnp.float32) for _ in range(_D)]
        + [pltpu.VMEM_SHARED((_NPAD, _F), jnp.float32)]
        + [pltpu.SemaphoreType.DMA] * (2 * _D)
    )

    def body(table_r, esd_r, out_r, *scr):
        idx = scr[0:_D]
        rows = scr[_D:2 * _D]
        acc = scr[2 * _D]
        sem_i = scr[2 * _D + 1:3 * _D + 1]
        sem_g = scr[3 * _D + 1:4 * _D + 1]
        c = lax.axis_index("c")
        s = lax.axis_index("s")
        wid = s * _NC + c

        zero16 = jnp.zeros((16,), jnp.float32)

        def zero_rows(i, carry):
            for j in range(_F // 16):
                rows[0][i, pl.ds(j * 16, 16)] = zero16
            return carry
        lax.fori_loop(0, _CH, zero_rows, 0)
        for r in range(_RPT // _CH):
            pltpu.sync_copy(rows[0], acc.at[pl.ds(s * _RPT + r * _CH, _CH)])

        plsc.subcore_barrier()

        def wait_i(b):
            pltpu.make_async_copy(esd_r.at[wid, 0], idx[b], sem_i[b]).wait()

        def wait_g(b):
            pltpu.make_async_copy(table_r.at[pl.ds(0, _CH)], rows[b],
                                  sem_g[b]).wait()

        for b in range(_D):
            pltpu.async_copy(esd_r.at[wid, b], idx[b], sem_i[b])
        for b in range(_D - 1):
            wait_i(b)
            pltpu.async_copy(table_r.at[idx[b].at[0]], rows[b], sem_g[b])

        def chunk4(i, carry):
            k0 = 4 * i
            for b in range(_D):
                k = k0 + b
                wait_g(b)
                pltpu.sync_copy(rows[b], acc.at[idx[b].at[1]], add=True)

                @pl.when(k + _D < _NCHUNK)
                def _():
                    pltpu.async_copy(esd_r.at[wid, k + _D], idx[b], sem_i[b])
                b3 = (b + _D - 1) % _D

                @pl.when(k + _D - 1 < _NCHUNK)
                def _():
                    wait_i(b3)
                    pltpu.async_copy(table_r.at[idx[b3].at[0]], rows[b3],
                                     sem_g[b3])
            return carry
        lax.fori_loop(0, _NCHUNK // _D, chunk4, 0)

        for b in range(_NCHUNK % _D):
            wait_g(b)
            pltpu.sync_copy(rows[b], acc.at[idx[b].at[1]], add=True)


        plsc.subcore_barrier()

        row0 = s * _RPT
        pltpu.sync_copy(acc.at[pl.ds(row0, _RPT)],
                        out_r.at[c, pl.ds(row0, _RPT)])

    return pl.kernel(body, out_type=out_type, mesh=mesh,
                     scratch_types=tuple(scratch))(table, esd)


def _sc_deg(dstp):
    mesh = plsc.VectorSubcoreMesh(core_axis_name="c", subcore_axis_name="s")
    out_type = jax.ShapeDtypeStruct((_NC, _NPAD, _F), jnp.float32)
    scratch = [
        pltpu.VMEM((_NCHD, _CHD), jnp.int32),
        pltpu.VMEM((_CHD, _F), jnp.float32),
        pltpu.VMEM_SHARED((_NPAD, _F), jnp.float32),
        pltpu.SemaphoreType.DMA,
    ]

    def body(dst_r, out_r, dst_v, ones_v, acc, sem):
        c = lax.axis_index("c")
        s = lax.axis_index("s")
        wid = s * _NC + c

        zero16 = jnp.zeros((16,), jnp.float32)

        def zero_rows(i, carry):
            for j in range(_F // 16):
                ones_v[i, pl.ds(j * 16, 16)] = zero16
            return carry
        lax.fori_loop(0, _CHD, zero_rows, 0)
        for r in range(_RPT // _CHD):
            pltpu.sync_copy(ones_v, acc.at[pl.ds(s * _RPT + r * _CHD, _CHD)])

        one16 = zero16 + 1.0

        def fill_ones(i, carry):
            for j in range(_F // 16):
                ones_v[i, pl.ds(j * 16, 16)] = one16
            return carry
        lax.fori_loop(0, _CHD, fill_ones, 0)

        pltpu.sync_copy(dst_r.at[wid], dst_v)

        plsc.subcore_barrier()

        def chunk(k, carry):
            pltpu.async_copy(ones_v, acc.at[dst_v.at[k]], sem, add=True)
            return carry
        lax.fori_loop(0, _NCHD, chunk, 0)

        def drain(k, carry):
            pltpu.make_async_copy(ones_v, acc.at[pl.ds(0, _CHD)], sem).wait()
            return carry
        lax.fori_loop(0, _NCHD, drain, 0)

        plsc.subcore_barrier()

        row0 = s * _RPT
        pltpu.sync_copy(acc.at[pl.ds(row0, _RPT)],
                        out_r.at[c, pl.ds(row0, _RPT)])

    return pl.kernel(body, out_type=out_type, mesh=mesh,
                     scratch_types=tuple(scratch))(dstp)


_BM = 1000
_NBLK = _N // _BM


def _dinv_from(deg_blk):
    deg = (deg_blk[0] + deg_blk[1])[:, 0:1]
    return jnp.where(deg > 0.0, lax.rsqrt(jnp.maximum(deg, 1e-12)), 0.0)


def _tc1a_body(x_r, a_r, w1_r, b1_r, w2_r, out_r):
    xa = x_r[...] + a_r[0] + a_r[1]
    h1 = jnp.maximum(
        jnp.dot(xa, w1_r[...], preferred_element_type=jnp.float32) + b1_r[...],
        0.0)
    out_r[...] = jnp.dot(h1, w2_r[...], preferred_element_type=jnp.float32)


def _tc1a(x, agg1, w1t, b1r, w2t):
    return pl.pallas_call(
        _tc1a_body,
        grid=(_NBLK,),
        in_specs=[
            pl.BlockSpec((_BM, _F), lambda i: (i, 0)),
            pl.BlockSpec((_NC, _BM, _F), lambda i: (0, i, 0)),
            pl.BlockSpec((_F, _H1), lambda i: (0, 0)),
            pl.BlockSpec((1, _H1), lambda i: (0, 0)),
            pl.BlockSpec((_H1, _F), lambda i: (0, 0)),
        ],
        out_specs=pl.BlockSpec((_BM, _F), lambda i: (i, 0)),
        out_shape=jax.ShapeDtypeStruct((_N, _F), jnp.float32),
    )(x, agg1, w1t, b1r, w2t)


def _tc1b_body(m_r, d_r, out_r):
    out_r[...] = m_r[...] * _dinv_from(d_r[...])


def _tc1b(m2, degp):
    return pl.pallas_call(
        _tc1b_body,
        grid=(_NBLK,),
        in_specs=[
            pl.BlockSpec((_BM, _F), lambda i: (i, 0)),
            pl.BlockSpec((_NC, _BM, _F), lambda i: (0, i, 0)),
        ],
        out_specs=pl.BlockSpec((_BM, _F), lambda i: (i, 0)),
        out_shape=jax.ShapeDtypeStruct((_N, _F), jnp.float32),
    )(m2, degp)


_BM2 = 1024


def _tc2_body(a_r, d_r, b2_r, w3_r, out_r):
    h = jnp.maximum((a_r[0] + a_r[1]) * _dinv_from(d_r[...]) + b2_r[...], 0.0)
    out_r[...] = jnp.dot(h, w3_r[...], preferred_element_type=jnp.float32)


def _tc2(agg2, degp, b2r, w3t_pad):
    return pl.pallas_call(
        _tc2_body,
        grid=(_NPAD // _BM2,),
        in_specs=[
            pl.BlockSpec((_NC, _BM2, _F), lambda i: (0, i, 0)),
            pl.BlockSpec((_NC, _BM2, _F), lambda i: (0, i, 0)),
            pl.BlockSpec((1, _F), lambda i: (0, 0)),
            pl.BlockSpec((_F, _F), lambda i: (0, 0)),
        ],
        out_specs=pl.BlockSpec((_BM2, _F), lambda i: (i, 0)),
        out_shape=jax.ShapeDtypeStruct((_NPAD, _F), jnp.float32),
    )(agg2, degp, b2r, w3t_pad)


def _tc3_body(a_r, m3_r, b3_r, bt_r, out_r):
    i = pl.program_id(0)
    h128 = jnp.maximum(a_r[0] + a_r[1] + m3_r[...] + b3_r[...], 0.0)
    h = h128[:, :_C]
    oh = (bt_r[...] == lax.broadcasted_iota(jnp.int32, (_BM, _G), 1)
          ).astype(jnp.float32)
    part = lax.dot_general(oh, h, (((0,), (0,)), ((), ())),
                           preferred_element_type=jnp.float32)

    @pl.when(i == 0)
    def _():
        out_r[...] = part

    @pl.when(i > 0)
    def _():
        out_r[...] += part

    @pl.when(i == _NBLK - 1)
    def _():
        p = out_r[...]
        m = jnp.max(p, axis=1, keepdims=True)
        lse = jnp.log(jnp.sum(jnp.exp(p - m), axis=1, keepdims=True)) + m
        out_r[...] = p - lse


def _tc3(agg3, m3, b3r_pad, batch_c):
    return pl.pallas_call(
        _tc3_body,
        grid=(_NBLK,),
        in_specs=[
            pl.BlockSpec((_NC, _BM, _F), lambda i: (0, i, 0)),
            pl.BlockSpec((_BM, _F), lambda i: (i, 0)),
            pl.BlockSpec((1, _F), lambda i: (0, 0)),
            pl.BlockSpec((_BM, 1), lambda i: (i, 0)),
        ],
        out_specs=pl.BlockSpec((_G, _C), lambda i: (0, 0)),
        out_shape=jax.ShapeDtypeStruct((_G, _C), jnp.float32),
    )(agg3, m3, b3r_pad, batch_c)


def kernel(x, edge_index, batch, W1, b1, W2, b2, W3, b3):
    esd = edge_index.reshape(2, _NW, _NCHUNK, _CH).transpose(1, 2, 0, 3)
    dstp = jnp.concatenate(
        [edge_index[1],
         jnp.full((_EPAD - _E,), _NPAD - 1, jnp.int32)]).reshape(
             _NW, _NCHD, _CHD)

    agg1 = _sc_agg(x, esd)
    degp = _sc_deg(dstp)
    m2 = _tc1a(x, agg1, W1.T, b1.reshape(1, -1), W2.T)
    m2s = _tc1b(m2, degp)
    agg2 = _sc_agg(m2s, esd)
    w3t_pad = jnp.zeros((_F, _F), jnp.float32).at[:, :_C].set(W3.T)
    m3 = _tc2(agg2, degp, b2.reshape(1, -1), w3t_pad)
    agg3 = _sc_agg(m3, esd)
    b3r_pad = jnp.zeros((1, _F), jnp.float32).at[:, :_C].set(b3.reshape(1, -1))
    return _tc3(agg3, m3, b3r_pad, batch.reshape(-1, 1))


# speedup vs baseline: 22.7056x; 1.0051x over previous
import jax
import jax.numpy as jnp
from jax import lax
from jax.experimental import pallas as pl
from jax.experimental.pallas import tpu as pltpu
from jax.experimental.pallas import tpu_sc as plsc

_N = 10000
_E = 320000
_F = 128
_H1 = 256
_C = 16
_G = 64

_NC = 2
_NS = 16
_NW = _NC * _NS
_EPW = _E // _NW
_CH = 80
_NCHUNK = _EPW // _CH
_NPAD = 10240
_RPT = _NPAD // _NS
_HR = _NPAD // _F


def _sc_agg(table, esd):
    mesh = plsc.VectorSubcoreMesh(core_axis_name="c", subcore_axis_name="s")
    out_type = jax.ShapeDtypeStruct((_NC, _NPAD, _F), jnp.float32)
    _D = 4
    scratch = (
        [pltpu.VMEM((2, _CH), jnp.int32) for _ in range(_D)]
        + [pltpu.VMEM((_CH, _F), jnp.float32) for _ in range(_D)]
        + [pltpu.VMEM_SHARED((_NPAD, _F), jnp.float32)]
        + [pltpu.SemaphoreType.DMA] * (2 * _D)
    )

    def body(table_r, esd_r, out_r, *scr):
        idx = scr[0:_D]
        rows = scr[_D:2 * _D]
        acc = scr[2 * _D]
        sem_i = scr[2 * _D + 1:3 * _D + 1]
        sem_g = scr[3 * _D + 1:4 * _D + 1]
        c = lax.axis_index("c")
        s = lax.axis_index("s")
        wid = s * _NC + c

        zero16 = jnp.zeros((16,), jnp.float32)

        def zero_rows(i, carry):
            for j in range(_F // 16):
                rows[0][i, pl.ds(j * 16, 16)] = zero16
            return carry
        lax.fori_loop(0, _CH, zero_rows, 0)
        for r in range(_RPT // _CH):
            pltpu.sync_copy(rows[0], acc.at[pl.ds(s * _RPT + r * _CH, _CH)])

        plsc.subcore_barrier()

        def wait_i(b):
            pltpu.make_async_copy(esd_r.at[wid, 0], idx[b], sem_i[b]).wait()

        def wait_g(b):
            pltpu.make_async_copy(table_r.at[pl.ds(0, _CH)], rows[b],
                                  sem_g[b]).wait()

        for b in range(_D):
            pltpu.async_copy(esd_r.at[wid, b], idx[b], sem_i[b])
        for b in range(_D - 1):
            wait_i(b)
            pltpu.async_copy(table_r.at[idx[b].at[0]], rows[b], sem_g[b])

        def chunk4(i, carry):
            k0 = 4 * i
            for b in range(_D):
                k = k0 + b
                wait_g(b)
                pltpu.sync_copy(rows[b], acc.at[idx[b].at[1]], add=True)

                @pl.when(k + _D < _NCHUNK)
                def _():
                    pltpu.async_copy(esd_r.at[wid, k + _D], idx[b], sem_i[b])
                b3 = (b + _D - 1) % _D

                @pl.when(k + _D - 1 < _NCHUNK)
                def _():
                    wait_i(b3)
                    pltpu.async_copy(table_r.at[idx[b3].at[0]], rows[b3],
                                     sem_g[b3])
            return carry
        lax.fori_loop(0, _NCHUNK // _D, chunk4, 0)

        for b in range(_NCHUNK % _D):
            wait_g(b)
            pltpu.sync_copy(rows[b], acc.at[idx[b].at[1]], add=True)


        plsc.subcore_barrier()

        row0 = s * _RPT
        pltpu.sync_copy(acc.at[pl.ds(row0, _RPT)],
                        out_r.at[c, pl.ds(row0, _RPT)])

    return pl.kernel(body, out_type=out_type, mesh=mesh,
                     scratch_types=tuple(scratch))(table, esd)


def _sc_deg(dst3):
    mesh = plsc.VectorSubcoreMesh(core_axis_name="c", subcore_axis_name="s")
    out_type = jax.ShapeDtypeStruct((_NC, _NPAD, _F), jnp.float32)
    scratch = [
        pltpu.VMEM((_NCHUNK, _CH), jnp.int32),
        pltpu.VMEM((_CH, _F), jnp.float32),
        pltpu.VMEM_SHARED((_NPAD, _F), jnp.float32),
        pltpu.SemaphoreType.DMA,
    ]

    def body(dst_r, out_r, dst_v, ones_v, acc, sem):
        c = lax.axis_index("c")
        s = lax.axis_index("s")
        wid = s * _NC + c

        zero16 = jnp.zeros((16,), jnp.float32)

        def zero_rows(i, carry):
            for j in range(_F // 16):
                ones_v[i, pl.ds(j * 16, 16)] = zero16
            return carry
        lax.fori_loop(0, _CH, zero_rows, 0)
        for r in range(_RPT // _CH):
            pltpu.sync_copy(ones_v, acc.at[pl.ds(s * _RPT + r * _CH, _CH)])

        one16 = zero16 + 1.0

        def fill_ones(i, carry):
            for j in range(_F // 16):
                ones_v[i, pl.ds(j * 16, 16)] = one16
            return carry
        lax.fori_loop(0, _CH, fill_ones, 0)

        pltpu.sync_copy(dst_r.at[wid], dst_v)

        plsc.subcore_barrier()

        def chunk(k, carry):
            pltpu.async_copy(ones_v, acc.at[dst_v.at[k]], sem, add=True)
            return carry
        lax.fori_loop(0, _NCHUNK, chunk, 0)

        def drain(k, carry):
            pltpu.make_async_copy(ones_v, acc.at[pl.ds(0, _CH)], sem).wait()
            return carry
        lax.fori_loop(0, _NCHUNK, drain, 0)

        plsc.subcore_barrier()

        row0 = s * _RPT
        pltpu.sync_copy(acc.at[pl.ds(row0, _RPT)],
                        out_r.at[c, pl.ds(row0, _RPT)])

    return pl.kernel(body, out_type=out_type, mesh=mesh,
                     scratch_types=tuple(scratch))(dst3)


_BM = 1000
_NBLK = _N // _BM


def _dinv_from(deg_blk):
    deg = (deg_blk[0] + deg_blk[1])[:, 0:1]
    return jnp.where(deg > 0.0, lax.rsqrt(jnp.maximum(deg, 1e-12)), 0.0)


def _tc1a_body(x_r, a_r, w1_r, b1_r, w2_r, out_r):
    xa = x_r[...] + a_r[0] + a_r[1]
    h1 = jnp.maximum(
        jnp.dot(xa, w1_r[...], preferred_element_type=jnp.float32) + b1_r[...],
        0.0)
    out_r[...] = jnp.dot(h1, w2_r[...], preferred_element_type=jnp.float32)


def _tc1a(x, agg1, w1t, b1r, w2t):
    return pl.pallas_call(
        _tc1a_body,
        grid=(_NBLK,),
        in_specs=[
            pl.BlockSpec((_BM, _F), lambda i: (i, 0)),
            pl.BlockSpec((_NC, _BM, _F), lambda i: (0, i, 0)),
            pl.BlockSpec((_F, _H1), lambda i: (0, 0)),
            pl.BlockSpec((1, _H1), lambda i: (0, 0)),
            pl.BlockSpec((_H1, _F), lambda i: (0, 0)),
        ],
        out_specs=pl.BlockSpec((_BM, _F), lambda i: (i, 0)),
        out_shape=jax.ShapeDtypeStruct((_N, _F), jnp.float32),
    )(x, agg1, w1t, b1r, w2t)


def _tc1b_body(m_r, d_r, out_r):
    out_r[...] = m_r[...] * _dinv_from(d_r[...])


def _tc1b(m2, degp):
    return pl.pallas_call(
        _tc1b_body,
        grid=(_NBLK,),
        in_specs=[
            pl.BlockSpec((_BM, _F), lambda i: (i, 0)),
            pl.BlockSpec((_NC, _BM, _F), lambda i: (0, i, 0)),
        ],
        out_specs=pl.BlockSpec((_BM, _F), lambda i: (i, 0)),
        out_shape=jax.ShapeDtypeStruct((_N, _F), jnp.float32),
    )(m2, degp)


_BM2 = 1024


def _tc2_body(a_r, d_r, b2_r, w3_r, out_r):
    h = jnp.maximum((a_r[0] + a_r[1]) * _dinv_from(d_r[...]) + b2_r[...], 0.0)
    out_r[...] = jnp.dot(h, w3_r[...], preferred_element_type=jnp.float32)


def _tc2(agg2, degp, b2r, w3t_pad):
    return pl.pallas_call(
        _tc2_body,
        grid=(_NPAD // _BM2,),
        in_specs=[
            pl.BlockSpec((_NC, _BM2, _F), lambda i: (0, i, 0)),
            pl.BlockSpec((_NC, _BM2, _F), lambda i: (0, i, 0)),
            pl.BlockSpec((1, _F), lambda i: (0, 0)),
            pl.BlockSpec((_F, _F), lambda i: (0, 0)),
        ],
        out_specs=pl.BlockSpec((_BM2, _F), lambda i: (i, 0)),
        out_shape=jax.ShapeDtypeStruct((_NPAD, _F), jnp.float32),
    )(agg2, degp, b2r, w3t_pad)


def _tc3_body(a_r, m3_r, b3_r, bt_r, out_r):
    i = pl.program_id(0)
    h128 = jnp.maximum(a_r[0] + a_r[1] + m3_r[...] + b3_r[...], 0.0)
    h = h128[:, :_C]
    oh = (bt_r[...] == lax.broadcasted_iota(jnp.int32, (_BM, _G), 1)
          ).astype(jnp.float32)
    part = lax.dot_general(oh, h, (((0,), (0,)), ((), ())),
                           preferred_element_type=jnp.float32)

    @pl.when(i == 0)
    def _():
        out_r[...] = part

    @pl.when(i > 0)
    def _():
        out_r[...] += part

    @pl.when(i == _NBLK - 1)
    def _():
        p = out_r[...]
        m = jnp.max(p, axis=1, keepdims=True)
        lse = jnp.log(jnp.sum(jnp.exp(p - m), axis=1, keepdims=True)) + m
        out_r[...] = p - lse


def _tc3(agg3, m3, b3r_pad, batch_c):
    return pl.pallas_call(
        _tc3_body,
        grid=(_NBLK,),
        in_specs=[
            pl.BlockSpec((_NC, _BM, _F), lambda i: (0, i, 0)),
            pl.BlockSpec((_BM, _F), lambda i: (i, 0)),
            pl.BlockSpec((1, _F), lambda i: (0, 0)),
            pl.BlockSpec((_BM, 1), lambda i: (i, 0)),
        ],
        out_specs=pl.BlockSpec((_G, _C), lambda i: (0, 0)),
        out_shape=jax.ShapeDtypeStruct((_G, _C), jnp.float32),
    )(agg3, m3, b3r_pad, batch_c)


def kernel(x, edge_index, batch, W1, b1, W2, b2, W3, b3):
    esd = edge_index.reshape(2, _NW, _NCHUNK, _CH).transpose(1, 2, 0, 3)
    dst3 = edge_index[1].reshape(_NW, _NCHUNK, _CH)

    agg1 = _sc_agg(x, esd)
    degp = _sc_deg(dst3)
    m2 = _tc1a(x, agg1, W1.T, b1.reshape(1, -1), W2.T)
    m2s = _tc1b(m2, degp)
    agg2 = _sc_agg(m2s, esd)
    w3t_pad = jnp.zeros((_F, _F), jnp.float32).at[:, :_C].set(W3.T)
    m3 = _tc2(agg2, degp, b2.reshape(1, -1), w3t_pad)
    agg3 = _sc_agg(m3, esd)
    b3r_pad = jnp.zeros((1, _F), jnp.float32).at[:, :_C].set(b3.reshape(1, -1))
    return _tc3(agg3, m3, b3r_pad, batch.reshape(-1, 1))
